# Initial kernel scaffold; baseline (speedup 1.0000x reference)
import jax
import jax.numpy as jnp
from jax.experimental import pallas as pl


def kernel(x, edge_index, gene_to_pathway_map, W1, b1, W2, b2, Wp, bp, Wc1, bc1, Wc2, bc2):
    raise NotImplementedError("write your pallas kernel here")


# speedup vs baseline: 45.0573x; 45.0573x over previous
import functools

import jax
import jax.numpy as jnp
from jax import lax
from jax.experimental import pallas as pl
from jax.experimental.pallas import tpu as pltpu
from jax.experimental.pallas import tpu_sc as plsc

N = 50000
E = 800000
H = 64
NPW = 100
NCLS = 5

NC, NS, L = 2, 16, 16
NW = NC * NS
NPAD = 50176
EPAD = 802816
EPW = EPAD // NW
EPT = EPAD // NS
BQ = 3136
NBQ = EPW // BQ
KSTR = 7
NBATCH = EPT // (KSTR * 128)
BCOL = 6272
TCG = NPAD // BCOL

_mesh = plsc.VectorSubcoreMesh(core_axis_name="c", subcore_axis_name="s")
_sc_params = pltpu.CompilerParams(needs_layout_passes=False)
_sc_params_nt = pltpu.CompilerParams(needs_layout_passes=False,
                                     use_tc_tiling_on_sc=False)


@functools.partial(
    pl.kernel,
    out_type=jax.ShapeDtypeStruct((NW, NPAD), jnp.float32),
    mesh=_mesh,
    compiler_params=_sc_params,
    scratch_types=[
        pltpu.VMEM((BQ,), jnp.int32),
        pltpu.VMEM((NPAD,), jnp.float32),
    ],
)
def _sc_deg(dst_hbm, out_hbm, idx_v, acc_v):
    w = lax.axis_index("c") * NS + lax.axis_index("s")
    zeros16 = jnp.zeros((L,), jnp.float32)
    ones16 = jnp.ones((L,), jnp.float32)

    def zero_body(i, _):
        acc_v[pl.ds(i * L, L)] = zeros16
        return _

    lax.fori_loop(0, NPAD // L, zero_body, 0)
    base = w * EPW

    def chunk(q, _):
        pltpu.sync_copy(dst_hbm.at[pl.ds(base + q * BQ, BQ)], idx_v)

        def body(i, _):
            idx = idx_v[pl.ds(i * L, L)]
            plsc.addupdate_scatter(acc_v, [idx], ones16)
            return _

        lax.fori_loop(0, BQ // L, body, 0)
        return _

    lax.fori_loop(0, NBQ, chunk, 0)
    pltpu.sync_copy(acc_v, out_hbm.at[w])


@functools.partial(
    pl.kernel,
    out_type=jax.ShapeDtypeStruct((NW, NPAD), jnp.float32),
    mesh=_mesh,
    compiler_params=_sc_params,
    scratch_types=[
        pltpu.VMEM((BQ,), jnp.int32),
        pltpu.VMEM((BQ,), jnp.int32),
        pltpu.VMEM((NPAD,), jnp.float32),
        pltpu.VMEM((NPAD,), jnp.float32),
    ],
)
def _sc_t1(src_hbm, dst_hbm, xp_hbm, out_hbm, sidx_v, didx_v, xp_v, acc_v):
    w = lax.axis_index("c") * NS + lax.axis_index("s")
    zeros16 = jnp.zeros((L,), jnp.float32)
    pltpu.sync_copy(xp_hbm, xp_v)

    def zero_body(i, _):
        acc_v[pl.ds(i * L, L)] = zeros16
        return _

    lax.fori_loop(0, NPAD // L, zero_body, 0)
    base = w * EPW

    def chunk(q, _):
        pltpu.sync_copy(src_hbm.at[pl.ds(base + q * BQ, BQ)], sidx_v)
        pltpu.sync_copy(dst_hbm.at[pl.ds(base + q * BQ, BQ)], didx_v)

        def body(i, _):
            s_idx = sidx_v[pl.ds(i * L, L)]
            vals = plsc.load_gather(xp_v, [s_idx])
            d_idx = didx_v[pl.ds(i * L, L)]
            plsc.addupdate_scatter(acc_v, [d_idx], vals)
            return _

        lax.fori_loop(0, BQ // L, body, 0)
        return _

    lax.fori_loop(0, NBQ, chunk, 0)
    pltpu.sync_copy(acc_v, out_hbm.at[w])


@functools.partial(
    pl.kernel,
    out_type=jax.ShapeDtypeStruct((NC, NPAD, 32), jnp.float32),
    mesh=_mesh,
    compiler_params=_sc_params_nt,
    scratch_types=[
        pltpu.VMEM((KSTR, 128), jnp.int32),
        pltpu.VMEM((KSTR, 128), jnp.int32),
        [pltpu.VMEM((128, 32), jnp.float32) for _ in range(KSTR)],
        pltpu.VMEM_SHARED((NPAD, 32), jnp.float32),
        pltpu.SemaphoreType.DMA,
    ],
)
def _sc_rowsum(src2d_hbm, dst2d_hbm, gcat_hbm, zero_hbm, out_hbm,
               sidx_v, didx_v, rows_v, acc_sh, sem):
    c = lax.axis_index("c")
    s = lax.axis_index("s")

    @pl.when(s == 0)
    def _():
        pltpu.sync_copy(zero_hbm, acc_sh)

    plsc.subcore_barrier()
    coff = (c * NPAD).astype(jnp.int32)

    def batch(jb, _):
        rbase = s * (EPT // 128) + jb * KSTR
        pltpu.sync_copy(src2d_hbm.at[pl.ds(rbase, KSTR)], sidx_v)
        pltpu.sync_copy(dst2d_hbm.at[pl.ds(rbase, KSTR)], didx_v)
        for b in range(KSTR):
            for k in range(128 // L):
                sidx_v[b, pl.ds(k * L, L)] = sidx_v[b, pl.ds(k * L, L)] + coff
        gets = [
            pltpu.async_copy(gcat_hbm.at[sidx_v.at[b]], rows_v[b], sem)
            for b in range(KSTR)
        ]
        for d in gets:
            d.wait()
        puts = [
            pltpu.async_copy(rows_v[b], acc_sh.at[didx_v.at[b]], sem, add=True)
            for b in range(KSTR)
        ]
        for d in puts:
            d.wait()
        return _

    lax.fori_loop(0, NBATCH, batch, 0)
    plsc.subcore_barrier()
    rows_per_tile = NPAD // NS
    pltpu.sync_copy(
        acc_sh.at[pl.ds(s * rows_per_tile, rows_per_tile)],
        out_hbm.at[c, pl.ds(s * rows_per_tile, rows_per_tile)],
    )


def _tc_combine_body(p_ref, x_ref, dis_ref, xp_ref):
    deg = 1.0 + jnp.sum(p_ref[...], axis=0, keepdims=True)
    dis = 1.0 / jnp.sqrt(deg)
    dis_ref[...] = dis
    xp_ref[...] = dis * x_ref[...]


def _tc_combine(p, xflat):
    return pl.pallas_call(
        _tc_combine_body,
        grid=(TCG,),
        in_specs=[
            pl.BlockSpec((NW, BCOL), lambda i: (0, i)),
            pl.BlockSpec((1, BCOL), lambda i: (0, i)),
        ],
        out_specs=[
            pl.BlockSpec((1, BCOL), lambda i: (0, i)),
            pl.BlockSpec((1, BCOL), lambda i: (0, i)),
        ],
        out_shape=[
            jax.ShapeDtypeStruct((1, NPAD), jnp.float32),
            jax.ShapeDtypeStruct((1, NPAD), jnp.float32),
        ],
    )(p, xflat)


def _tc_dense_body(t_ref, xp_ref, dis_ref, w1_ref, b1_ref, w2_ref, g_ref):
    t1_raw = jnp.sum(t_ref[...], axis=0, keepdims=True) + xp_ref[...]
    t1 = dis_ref[...] * t1_raw
    h1p = lax.dot_general(t1, w1_ref[...], (((0,), (0,)), ((), ())),
                          preferred_element_type=jnp.float32)
    ones_row = jnp.ones((1, H), jnp.float32)
    d_mat = lax.dot_general(dis_ref[...], ones_row, (((0,), (0,)), ((), ())),
                            preferred_element_type=jnp.float32)
    h1 = jnp.maximum(h1p + b1_ref[...], 0.0)
    g = lax.dot_general(h1, w2_ref[...], (((1,), (0,)), ((), ())),
                        preferred_element_type=jnp.float32)
    gp = d_mat * g
    g_ref[0] = gp[:, :32]
    g_ref[1] = gp[:, 32:]


def _tc_dense(t, xp, dis, w1, b1, w2):
    return pl.pallas_call(
        _tc_dense_body,
        grid=(TCG,),
        in_specs=[
            pl.BlockSpec((NW, BCOL), lambda i: (0, i)),
            pl.BlockSpec((1, BCOL), lambda i: (0, i)),
            pl.BlockSpec((1, BCOL), lambda i: (0, i)),
            pl.BlockSpec((1, H), lambda i: (0, 0)),
            pl.BlockSpec((1, H), lambda i: (0, 0)),
            pl.BlockSpec((H, H), lambda i: (0, 0)),
        ],
        out_specs=pl.BlockSpec((NC, BCOL, 32), lambda i: (0, i, 0)),
        out_shape=jax.ShapeDtypeStruct((NC, NPAD, 32), jnp.float32),
    )(t, xp, dis, w1, b1, w2)


def _tc_final_body(acc_ref, g_ref, dis_ref, b2_ref, wp_ref, bp_ref,
                   wc1_ref, bc1_ref, wc2_ref, bc2_ref, out_ref, s_acc):
    i = pl.program_id(0)

    @pl.when(i == 0)
    def _():
        s_acc[...] = jnp.zeros_like(s_acc)

    ones_row = jnp.ones((1, H), jnp.float32)
    d_mat = lax.dot_general(dis_ref[...], ones_row, (((0,), (0,)), ((), ())),
                            preferred_element_type=jnp.float32)
    accf = jnp.concatenate([acc_ref[0], acc_ref[1]], axis=1)
    gf = jnp.concatenate([g_ref[0], g_ref[1]], axis=1)
    agg = d_mat * (accf + gf) + b2_ref[...]
    h2 = jnp.maximum(agg, 0.0)
    row = lax.broadcasted_iota(jnp.int32, (BCOL, H), 0) + i * BCOL
    h2 = jnp.where(row < N, h2, 0.0)
    pf = lax.dot_general(h2, wp_ref[...], (((1,), (0,)), ((), ())),
                         preferred_element_type=jnp.float32)
    s_acc[...] = s_acc[...] + jnp.sum(pf, axis=0, keepdims=True)

    @pl.when(i == TCG - 1)
    def _():
        pooled = s_acc[...] * (1.0 / N) + bp_ref[...]
        z = jnp.maximum(
            lax.dot_general(pooled, wc1_ref[...], (((1,), (0,)), ((), ())),
                            preferred_element_type=jnp.float32) + bc1_ref[...],
            0.0)
        out_ref[...] = lax.dot_general(
            z, wc2_ref[...], (((1,), (0,)), ((), ())),
            preferred_element_type=jnp.float32) + bc2_ref[...]


def _tc_final(acc, g, dis, b2, wp, bp, wc1, bc1, wc2, bc2):
    return pl.pallas_call(
        _tc_final_body,
        grid=(TCG,),
        in_specs=[
            pl.BlockSpec((NC, BCOL, 32), lambda i: (0, i, 0)),
            pl.BlockSpec((NC, BCOL, 32), lambda i: (0, i, 0)),
            pl.BlockSpec((1, BCOL), lambda i: (0, i)),
            pl.BlockSpec((1, H), lambda i: (0, 0)),
            pl.BlockSpec((H, NPW), lambda i: (0, 0)),
            pl.BlockSpec((1, NPW), lambda i: (0, 0)),
            pl.BlockSpec((NPW, 128), lambda i: (0, 0)),
            pl.BlockSpec((1, 128), lambda i: (0, 0)),
            pl.BlockSpec((128, NCLS), lambda i: (0, 0)),
            pl.BlockSpec((1, NCLS), lambda i: (0, 0)),
        ],
        out_specs=pl.BlockSpec((1, NCLS), lambda i: (0, 0)),
        out_shape=jax.ShapeDtypeStruct((1, NCLS), jnp.float32),
        scratch_shapes=[pltpu.VMEM((1, NPW), jnp.float32)],
    )(acc, g, dis, b2, wp, bp, wc1, bc1, wc2, bc2)


def kernel(x, edge_index, gene_to_pathway_map, W1, b1, W2, b2, Wp, bp,
           Wc1, bc1, Wc2, bc2):
    del gene_to_pathway_map
    src = edge_index[0]
    dst = edge_index[1]
    pad = EPAD - E
    src_p = jnp.concatenate([src, jnp.zeros((pad,), jnp.int32)])
    dst_p = jnp.concatenate([dst, jnp.full((pad,), N, jnp.int32)])
    src2d = src_p.reshape(EPAD // 128, 128)
    dst2d = dst_p.reshape(EPAD // 128, 128)
    xflat = jnp.concatenate([x[:, 0], jnp.zeros((NPAD - N,), jnp.float32)])
    xflat = xflat.reshape(1, NPAD)

    deg_parts = _sc_deg(dst_p)
    dis, xp = _tc_combine(deg_parts, xflat)
    t_parts = _sc_t1(src_p, dst_p, xp.reshape(NPAD))
    gstack = _tc_dense(t_parts, xp, dis, W1.reshape(1, H), b1.reshape(1, H),
                       W2)
    gcat = gstack.reshape(NC * NPAD, 32)
    zero_acc = jnp.zeros((NPAD, 32), jnp.float32)
    acc = _sc_rowsum(src2d, dst2d, gcat, zero_acc)
    out = _tc_final(acc, gstack, dis, b2.reshape(1, H), Wp, bp.reshape(1, NPW),
                    Wc1, bc1.reshape(1, 128), Wc2, bc2.reshape(1, NCLS))
    return out


# speedup vs baseline: 49.0410x; 1.0884x over previous
import functools

import jax
import jax.numpy as jnp
from jax import lax
from jax.experimental import pallas as pl
from jax.experimental.pallas import tpu as pltpu
from jax.experimental.pallas import tpu_sc as plsc

N = 50000
E = 800000
H = 64
NPW = 100
NCLS = 5

NC, NS, L = 2, 16, 16
NW = NC * NS
NPAD = 50176
EPAD = 802816
EPW = EPAD // NW
EPT = EPAD // NS
BQ = 3136
NBQ = EPW // BQ
KSTR = 7
NBATCH = EPT // (KSTR * 128)
BCOL = 6272
TCG = NPAD // BCOL

_mesh = plsc.VectorSubcoreMesh(core_axis_name="c", subcore_axis_name="s")
_sc_params = pltpu.CompilerParams(needs_layout_passes=False)
_sc_params_nt = pltpu.CompilerParams(needs_layout_passes=False,
                                     use_tc_tiling_on_sc=False)


@functools.partial(
    pl.kernel,
    out_type=jax.ShapeDtypeStruct((NW, NPAD), jnp.float32),
    mesh=_mesh,
    compiler_params=_sc_params,
    scratch_types=[
        pltpu.VMEM((BQ,), jnp.int32),
        pltpu.VMEM((NPAD,), jnp.float32),
    ],
)
def _sc_deg(dst_hbm, out_hbm, idx_v, acc_v):
    w = lax.axis_index("c") * NS + lax.axis_index("s")
    zeros16 = jnp.zeros((L,), jnp.float32)
    ones16 = jnp.ones((L,), jnp.float32)

    def zero_body(i, _):
        acc_v[pl.ds(i * L, L)] = zeros16
        return _

    lax.fori_loop(0, NPAD // L, zero_body, 0)
    base = w * EPW

    def chunk(q, _):
        pltpu.sync_copy(dst_hbm.at[pl.ds(base + q * BQ, BQ)], idx_v)

        def body(i, _):
            idx = idx_v[pl.ds(i * L, L)]
            plsc.addupdate_scatter(acc_v, [idx], ones16)
            return _

        lax.fori_loop(0, BQ // L, body, 0)
        return _

    lax.fori_loop(0, NBQ, chunk, 0)
    pltpu.sync_copy(acc_v, out_hbm.at[w])


@functools.partial(
    pl.kernel,
    out_type=jax.ShapeDtypeStruct((NW, NPAD), jnp.float32),
    mesh=_mesh,
    compiler_params=_sc_params,
    scratch_types=[
        pltpu.VMEM((BQ,), jnp.int32),
        pltpu.VMEM((BQ,), jnp.int32),
        pltpu.VMEM((NPAD,), jnp.float32),
        pltpu.VMEM((NPAD,), jnp.float32),
    ],
)
def _sc_t1(src_hbm, dst_hbm, xp_hbm, out_hbm, sidx_v, didx_v, xp_v, acc_v):
    w = lax.axis_index("c") * NS + lax.axis_index("s")
    zeros16 = jnp.zeros((L,), jnp.float32)
    pltpu.sync_copy(xp_hbm, xp_v)

    def zero_body(i, _):
        acc_v[pl.ds(i * L, L)] = zeros16
        return _

    lax.fori_loop(0, NPAD // L, zero_body, 0)
    base = w * EPW

    def chunk(q, _):
        pltpu.sync_copy(src_hbm.at[pl.ds(base + q * BQ, BQ)], sidx_v)
        pltpu.sync_copy(dst_hbm.at[pl.ds(base + q * BQ, BQ)], didx_v)

        def body(i, _):
            s_idx = sidx_v[pl.ds(i * L, L)]
            vals = plsc.load_gather(xp_v, [s_idx])
            d_idx = didx_v[pl.ds(i * L, L)]
            plsc.addupdate_scatter(acc_v, [d_idx], vals)
            return _

        lax.fori_loop(0, BQ // L, body, 0)
        return _

    lax.fori_loop(0, NBQ, chunk, 0)
    pltpu.sync_copy(acc_v, out_hbm.at[w])


@functools.partial(
    pl.kernel,
    out_type=jax.ShapeDtypeStruct((NC, NPAD, 32), jnp.bfloat16),
    mesh=_mesh,
    compiler_params=_sc_params_nt,
    scratch_types=[
        pltpu.VMEM((KSTR, 128), jnp.int32),
        pltpu.VMEM((KSTR, 128), jnp.int32),
        [pltpu.VMEM((128, 32), jnp.bfloat16) for _ in range(KSTR)],
        pltpu.VMEM_SHARED((NPAD, 32), jnp.bfloat16),
        pltpu.SemaphoreType.DMA,
    ],
)
def _sc_rowsum(src2d_hbm, dst2d_hbm, gcat_hbm, zero_hbm, out_hbm,
               sidx_v, didx_v, rows_v, acc_sh, sem):
    c = lax.axis_index("c")
    s = lax.axis_index("s")

    @pl.when(s == 0)
    def _():
        pltpu.sync_copy(zero_hbm, acc_sh)

    plsc.subcore_barrier()
    coff = (c * NPAD).astype(jnp.int32)

    def batch(jb, _):
        rbase = s * (EPT // 128) + jb * KSTR
        pltpu.sync_copy(src2d_hbm.at[pl.ds(rbase, KSTR)], sidx_v)
        pltpu.sync_copy(dst2d_hbm.at[pl.ds(rbase, KSTR)], didx_v)
        for b in range(KSTR):
            for q in range(128 // L):
                sidx_v[b, pl.ds(q * L, L)] = sidx_v[b, pl.ds(q * L, L)] + coff
        gets = [
            pltpu.async_copy(gcat_hbm.at[sidx_v.at[b]], rows_v[b], sem)
            for b in range(KSTR)
        ]
        for d in gets:
            d.wait()
        puts = [
            pltpu.async_copy(rows_v[b], acc_sh.at[didx_v.at[b]], sem, add=True)
            for b in range(KSTR)
        ]
        for d in puts:
            d.wait()
        return _

    lax.fori_loop(0, NBATCH, batch, 0)
    plsc.subcore_barrier()
    rows_per_tile = NPAD // NS
    pltpu.sync_copy(
        acc_sh.at[pl.ds(s * rows_per_tile, rows_per_tile)],
        out_hbm.at[c, pl.ds(s * rows_per_tile, rows_per_tile)],
    )


def _tc_combine_body(p_ref, x_ref, dis_ref, xp_ref):
    deg = 1.0 + jnp.sum(p_ref[...], axis=0, keepdims=True)
    dis = 1.0 / jnp.sqrt(deg)
    dis_ref[...] = dis
    xp_ref[...] = dis * x_ref[...]


def _tc_combine(p, xflat):
    return pl.pallas_call(
        _tc_combine_body,
        grid=(TCG,),
        in_specs=[
            pl.BlockSpec((NW, BCOL), lambda i: (0, i)),
            pl.BlockSpec((1, BCOL), lambda i: (0, i)),
        ],
        out_specs=[
            pl.BlockSpec((1, BCOL), lambda i: (0, i)),
            pl.BlockSpec((1, BCOL), lambda i: (0, i)),
        ],
        out_shape=[
            jax.ShapeDtypeStruct((1, NPAD), jnp.float32),
            jax.ShapeDtypeStruct((1, NPAD), jnp.float32),
        ],
    )(p, xflat)


def _tc_dense_body(t_ref, xp_ref, dis_ref, w1_ref, b1_ref, w2_ref, g_ref):
    t1_raw = jnp.sum(t_ref[...], axis=0, keepdims=True) + xp_ref[...]
    t1 = dis_ref[...] * t1_raw
    h1p = lax.dot_general(t1, w1_ref[...], (((0,), (0,)), ((), ())),
                          preferred_element_type=jnp.float32)
    ones_row = jnp.ones((1, H), jnp.float32)
    d_mat = lax.dot_general(dis_ref[...], ones_row, (((0,), (0,)), ((), ())),
                            preferred_element_type=jnp.float32)
    h1 = jnp.maximum(h1p + b1_ref[...], 0.0)
    g = lax.dot_general(h1, w2_ref[...], (((1,), (0,)), ((), ())),
                        preferred_element_type=jnp.float32)
    gp = (d_mat * g).astype(jnp.bfloat16)
    g_ref[0] = gp[:, :32]
    g_ref[1] = gp[:, 32:]


def _tc_dense(t, xp, dis, w1, b1, w2):
    return pl.pallas_call(
        _tc_dense_body,
        grid=(TCG,),
        in_specs=[
            pl.BlockSpec((NW, BCOL), lambda i: (0, i)),
            pl.BlockSpec((1, BCOL), lambda i: (0, i)),
            pl.BlockSpec((1, BCOL), lambda i: (0, i)),
            pl.BlockSpec((1, H), lambda i: (0, 0)),
            pl.BlockSpec((1, H), lambda i: (0, 0)),
            pl.BlockSpec((H, H), lambda i: (0, 0)),
        ],
        out_specs=pl.BlockSpec((NC, BCOL, 32), lambda i: (0, i, 0)),
        out_shape=jax.ShapeDtypeStruct((NC, NPAD, 32), jnp.bfloat16),
    )(t, xp, dis, w1, b1, w2)


def _tc_final_body(acc_ref, g_ref, dis_ref, b2_ref, wp_ref, bp_ref,
                   wc1_ref, bc1_ref, wc2_ref, bc2_ref, out_ref, s_acc):
    i = pl.program_id(0)

    @pl.when(i == 0)
    def _():
        s_acc[...] = jnp.zeros_like(s_acc)

    ones_row = jnp.ones((1, H), jnp.float32)
    d_mat = lax.dot_general(dis_ref[...], ones_row, (((0,), (0,)), ((), ())),
                            preferred_element_type=jnp.float32)
    accf = jnp.concatenate([acc_ref[0], acc_ref[1]],
                           axis=1).astype(jnp.float32)
    gf = jnp.concatenate([g_ref[0], g_ref[1]],
                         axis=1).astype(jnp.float32)
    agg = d_mat * (accf + gf) + b2_ref[...]
    h2 = jnp.maximum(agg, 0.0)
    row = lax.broadcasted_iota(jnp.int32, (BCOL, H), 0) + i * BCOL
    h2 = jnp.where(row < N, h2, 0.0)
    pf = lax.dot_general(h2, wp_ref[...], (((1,), (0,)), ((), ())),
                         preferred_element_type=jnp.float32)
    s_acc[...] = s_acc[...] + jnp.sum(pf, axis=0, keepdims=True)

    @pl.when(i == TCG - 1)
    def _():
        pooled = s_acc[...] * (1.0 / N) + bp_ref[...]
        z = jnp.maximum(
            lax.dot_general(pooled, wc1_ref[...], (((1,), (0,)), ((), ())),
                            preferred_element_type=jnp.float32) + bc1_ref[...],
            0.0)
        out_ref[...] = lax.dot_general(
            z, wc2_ref[...], (((1,), (0,)), ((), ())),
            preferred_element_type=jnp.float32) + bc2_ref[...]


def _tc_final(acc, g, dis, b2, wp, bp, wc1, bc1, wc2, bc2):
    return pl.pallas_call(
        _tc_final_body,
        grid=(TCG,),
        in_specs=[
            pl.BlockSpec((NC, BCOL, 32), lambda i: (0, i, 0)),
            pl.BlockSpec((NC, BCOL, 32), lambda i: (0, i, 0)),
            pl.BlockSpec((1, BCOL), lambda i: (0, i)),
            pl.BlockSpec((1, H), lambda i: (0, 0)),
            pl.BlockSpec((H, NPW), lambda i: (0, 0)),
            pl.BlockSpec((1, NPW), lambda i: (0, 0)),
            pl.BlockSpec((NPW, 128), lambda i: (0, 0)),
            pl.BlockSpec((1, 128), lambda i: (0, 0)),
            pl.BlockSpec((128, NCLS), lambda i: (0, 0)),
            pl.BlockSpec((1, NCLS), lambda i: (0, 0)),
        ],
        out_specs=pl.BlockSpec((1, NCLS), lambda i: (0, 0)),
        out_shape=jax.ShapeDtypeStruct((1, NCLS), jnp.float32),
        scratch_shapes=[pltpu.VMEM((1, NPW), jnp.float32)],
    )(acc, g, dis, b2, wp, bp, wc1, bc1, wc2, bc2)


def kernel(x, edge_index, gene_to_pathway_map, W1, b1, W2, b2, Wp, bp,
           Wc1, bc1, Wc2, bc2):
    del gene_to_pathway_map
    src = edge_index[0]
    dst = edge_index[1]
    pad = EPAD - E
    src_p = jnp.concatenate([src, jnp.zeros((pad,), jnp.int32)])
    dst_p = jnp.concatenate([dst, jnp.full((pad,), N, jnp.int32)])
    src2d = src_p.reshape(EPAD // 128, 128)
    dst2d = dst_p.reshape(EPAD // 128, 128)
    xflat = jnp.concatenate([x[:, 0], jnp.zeros((NPAD - N,), jnp.float32)])
    xflat = xflat.reshape(1, NPAD)

    deg_parts = _sc_deg(dst_p)
    dis, xp = _tc_combine(deg_parts, xflat)
    t_parts = _sc_t1(src_p, dst_p, xp.reshape(NPAD))
    gstack = _tc_dense(t_parts, xp, dis, W1.reshape(1, H), b1.reshape(1, H),
                       W2)
    gcat = gstack.reshape(NC * NPAD, 32)
    zero_acc = jnp.zeros((NPAD, 32), jnp.bfloat16)
    acc = _sc_rowsum(src2d, dst2d, gcat, zero_acc)
    out = _tc_final(acc, gstack, dis, b2.reshape(1, H), Wp, bp.reshape(1, NPW),
                    Wc1, bc1.reshape(1, 128), Wc2, bc2.reshape(1, NCLS))
    return out


# speedup vs baseline: 55.1727x; 1.1250x over previous
import functools

import jax
import jax.numpy as jnp
from jax import lax
from jax.experimental import pallas as pl
from jax.experimental.pallas import tpu as pltpu
from jax.experimental.pallas import tpu_sc as plsc

N = 50000
E = 800000
H = 64
NPW = 100
NCLS = 5

NC, NS, L = 2, 16, 16
NW = NC * NS
NPAD = 50176
EPAD = 802816
EPW = EPAD // NW
EPT = EPAD // NS
BQ = 3136
NBQ = EPW // BQ
KSTR = 14
NBATCH = EPT // (KSTR * 128)
BCOL = 6272
TCG = NPAD // BCOL

_mesh = plsc.VectorSubcoreMesh(core_axis_name="c", subcore_axis_name="s")
_sc_params = pltpu.CompilerParams(needs_layout_passes=False)
_sc_params_nt = pltpu.CompilerParams(needs_layout_passes=False,
                                     use_tc_tiling_on_sc=False)


@functools.partial(
    pl.kernel,
    out_type=jax.ShapeDtypeStruct((NW, NPAD), jnp.float32),
    mesh=_mesh,
    compiler_params=_sc_params,
    scratch_types=[
        pltpu.VMEM((BQ,), jnp.int32),
        pltpu.VMEM((NPAD,), jnp.float32),
    ],
)
def _sc_deg(dst_hbm, zf_hbm, out_hbm, idx_v, acc_v):
    w = lax.axis_index("c") * NS + lax.axis_index("s")
    ones16 = jnp.ones((L,), jnp.float32)
    pltpu.sync_copy(zf_hbm, acc_v)
    base = w * EPW

    def chunk(q, _):
        pltpu.sync_copy(dst_hbm.at[pl.ds(base + q * BQ, BQ)], idx_v)

        def body(i, _):
            idx = idx_v[pl.ds(i * L, L)]
            plsc.addupdate_scatter(acc_v, [idx], ones16)
            return _

        lax.fori_loop(0, BQ // L, body, 0, unroll=8)
        return _

    lax.fori_loop(0, NBQ, chunk, 0)
    pltpu.sync_copy(acc_v, out_hbm.at[w])


@functools.partial(
    pl.kernel,
    out_type=jax.ShapeDtypeStruct((NW, NPAD), jnp.float32),
    mesh=_mesh,
    compiler_params=_sc_params,
    scratch_types=[
        pltpu.VMEM((BQ,), jnp.int32),
        pltpu.VMEM((BQ,), jnp.int32),
        pltpu.VMEM((NPAD,), jnp.float32),
        pltpu.VMEM((NPAD,), jnp.float32),
    ],
)
def _sc_t1(src_hbm, dst_hbm, xp_hbm, zf_hbm, out_hbm, sidx_v, didx_v, xp_v,
           acc_v):
    w = lax.axis_index("c") * NS + lax.axis_index("s")
    pltpu.sync_copy(xp_hbm, xp_v)
    pltpu.sync_copy(zf_hbm, acc_v)
    base = w * EPW

    def chunk(q, _):
        pltpu.sync_copy(src_hbm.at[pl.ds(base + q * BQ, BQ)], sidx_v)
        pltpu.sync_copy(dst_hbm.at[pl.ds(base + q * BQ, BQ)], didx_v)

        def body(i, _):
            s_idx = sidx_v[pl.ds(i * L, L)]
            vals = plsc.load_gather(xp_v, [s_idx])
            d_idx = didx_v[pl.ds(i * L, L)]
            plsc.addupdate_scatter(acc_v, [d_idx], vals)
            return _

        lax.fori_loop(0, BQ // L, body, 0, unroll=8)
        return _

    lax.fori_loop(0, NBQ, chunk, 0)
    pltpu.sync_copy(acc_v, out_hbm.at[w])


@functools.partial(
    pl.kernel,
    out_type=jax.ShapeDtypeStruct((NC, NPAD, 32), jnp.bfloat16),
    mesh=_mesh,
    compiler_params=_sc_params_nt,
    scratch_types=[
        pltpu.VMEM((KSTR, 128), jnp.int32),
        pltpu.VMEM((KSTR, 128), jnp.int32),
        [pltpu.VMEM((128, 32), jnp.bfloat16) for _ in range(KSTR)],
        pltpu.VMEM_SHARED((NPAD, 32), jnp.bfloat16),
        pltpu.SemaphoreType.DMA,
    ],
)
def _sc_rowsum(src2d_hbm, dst2d_hbm, gcat_hbm, zero_hbm, out_hbm,
               sidx_v, didx_v, rows_v, acc_sh, sem):
    c = lax.axis_index("c")
    s = lax.axis_index("s")

    @pl.when(s == 0)
    def _():
        pltpu.sync_copy(zero_hbm, acc_sh)

    plsc.subcore_barrier()
    coff = (c * NPAD).astype(jnp.int32)

    def batch(jb, _):
        rbase = s * (EPT // 128) + jb * KSTR
        pltpu.sync_copy(src2d_hbm.at[pl.ds(rbase, KSTR)], sidx_v)
        pltpu.sync_copy(dst2d_hbm.at[pl.ds(rbase, KSTR)], didx_v)
        for b in range(KSTR):
            for q in range(128 // L):
                sidx_v[b, pl.ds(q * L, L)] = sidx_v[b, pl.ds(q * L, L)] + coff
        gets = [
            pltpu.async_copy(gcat_hbm.at[sidx_v.at[b]], rows_v[b], sem)
            for b in range(KSTR)
        ]
        for d in gets:
            d.wait()
        puts = [
            pltpu.async_copy(rows_v[b], acc_sh.at[didx_v.at[b]], sem, add=True)
            for b in range(KSTR)
        ]
        for d in puts:
            d.wait()
        return _

    lax.fori_loop(0, NBATCH, batch, 0)
    plsc.subcore_barrier()
    rows_per_tile = NPAD // NS
    pltpu.sync_copy(
        acc_sh.at[pl.ds(s * rows_per_tile, rows_per_tile)],
        out_hbm.at[c, pl.ds(s * rows_per_tile, rows_per_tile)],
    )


def _tc_combine_body(p_ref, x_ref, dis_ref, xp_ref):
    deg = 1.0 + jnp.sum(p_ref[...], axis=0, keepdims=True)
    dis = 1.0 / jnp.sqrt(deg)
    dis_ref[...] = dis
    xp_ref[...] = dis * x_ref[...]


def _tc_combine(p, xflat):
    return pl.pallas_call(
        _tc_combine_body,
        grid=(TCG,),
        in_specs=[
            pl.BlockSpec((NW, BCOL), lambda i: (0, i)),
            pl.BlockSpec((1, BCOL), lambda i: (0, i)),
        ],
        out_specs=[
            pl.BlockSpec((1, BCOL), lambda i: (0, i)),
            pl.BlockSpec((1, BCOL), lambda i: (0, i)),
        ],
        out_shape=[
            jax.ShapeDtypeStruct((1, NPAD), jnp.float32),
            jax.ShapeDtypeStruct((1, NPAD), jnp.float32),
        ],
    )(p, xflat)


def _tc_dense_body(t_ref, xp_ref, dis_ref, w1_ref, b1_ref, w2_ref, g_ref):
    t1_raw = jnp.sum(t_ref[...], axis=0, keepdims=True) + xp_ref[...]
    t1 = dis_ref[...] * t1_raw
    h1p = lax.dot_general(t1, w1_ref[...], (((0,), (0,)), ((), ())),
                          preferred_element_type=jnp.float32)
    ones_row = jnp.ones((1, H), jnp.float32)
    d_mat = lax.dot_general(dis_ref[...], ones_row, (((0,), (0,)), ((), ())),
                            preferred_element_type=jnp.float32)
    h1 = jnp.maximum(h1p + b1_ref[...], 0.0)
    g = lax.dot_general(h1, w2_ref[...], (((1,), (0,)), ((), ())),
                        preferred_element_type=jnp.float32)
    gp = (d_mat * g).astype(jnp.bfloat16)
    g_ref[0] = gp[:, :32]
    g_ref[1] = gp[:, 32:]


def _tc_dense(t, xp, dis, w1, b1, w2):
    return pl.pallas_call(
        _tc_dense_body,
        grid=(TCG,),
        in_specs=[
            pl.BlockSpec((NW, BCOL), lambda i: (0, i)),
            pl.BlockSpec((1, BCOL), lambda i: (0, i)),
            pl.BlockSpec((1, BCOL), lambda i: (0, i)),
            pl.BlockSpec((1, H), lambda i: (0, 0)),
            pl.BlockSpec((1, H), lambda i: (0, 0)),
            pl.BlockSpec((H, H), lambda i: (0, 0)),
        ],
        out_specs=pl.BlockSpec((NC, BCOL, 32), lambda i: (0, i, 0)),
        out_shape=jax.ShapeDtypeStruct((NC, NPAD, 32), jnp.bfloat16),
    )(t, xp, dis, w1, b1, w2)


def _tc_final_body(acc_ref, g_ref, dis_ref, b2_ref, wp_ref, bp_ref,
                   wc1_ref, bc1_ref, wc2_ref, bc2_ref, out_ref, s_acc):
    i = pl.program_id(0)

    @pl.when(i == 0)
    def _():
        s_acc[...] = jnp.zeros_like(s_acc)

    ones_row = jnp.ones((1, H), jnp.float32)
    d_mat = lax.dot_general(dis_ref[...], ones_row, (((0,), (0,)), ((), ())),
                            preferred_element_type=jnp.float32)
    accf = jnp.concatenate([acc_ref[0], acc_ref[1]],
                           axis=1).astype(jnp.float32)
    gf = jnp.concatenate([g_ref[0], g_ref[1]],
                         axis=1).astype(jnp.float32)
    agg = d_mat * (accf + gf) + b2_ref[...]
    h2 = jnp.maximum(agg, 0.0)
    row = lax.broadcasted_iota(jnp.int32, (BCOL, H), 0) + i * BCOL
    h2 = jnp.where(row < N, h2, 0.0)
    pf = lax.dot_general(h2, wp_ref[...], (((1,), (0,)), ((), ())),
                         preferred_element_type=jnp.float32)
    s_acc[...] = s_acc[...] + jnp.sum(pf, axis=0, keepdims=True)

    @pl.when(i == TCG - 1)
    def _():
        pooled = s_acc[...] * (1.0 / N) + bp_ref[...]
        z = jnp.maximum(
            lax.dot_general(pooled, wc1_ref[...], (((1,), (0,)), ((), ())),
                            preferred_element_type=jnp.float32) + bc1_ref[...],
            0.0)
        out_ref[...] = lax.dot_general(
            z, wc2_ref[...], (((1,), (0,)), ((), ())),
            preferred_element_type=jnp.float32) + bc2_ref[...]


def _tc_final(acc, g, dis, b2, wp, bp, wc1, bc1, wc2, bc2):
    return pl.pallas_call(
        _tc_final_body,
        grid=(TCG,),
        in_specs=[
            pl.BlockSpec((NC, BCOL, 32), lambda i: (0, i, 0)),
            pl.BlockSpec((NC, BCOL, 32), lambda i: (0, i, 0)),
            pl.BlockSpec((1, BCOL), lambda i: (0, i)),
            pl.BlockSpec((1, H), lambda i: (0, 0)),
            pl.BlockSpec((H, NPW), lambda i: (0, 0)),
            pl.BlockSpec((1, NPW), lambda i: (0, 0)),
            pl.BlockSpec((NPW, 128), lambda i: (0, 0)),
            pl.BlockSpec((1, 128), lambda i: (0, 0)),
            pl.BlockSpec((128, NCLS), lambda i: (0, 0)),
            pl.BlockSpec((1, NCLS), lambda i: (0, 0)),
        ],
        out_specs=pl.BlockSpec((1, NCLS), lambda i: (0, 0)),
        out_shape=jax.ShapeDtypeStruct((1, NCLS), jnp.float32),
        scratch_shapes=[pltpu.VMEM((1, NPW), jnp.float32)],
    )(acc, g, dis, b2, wp, bp, wc1, bc1, wc2, bc2)


def kernel(x, edge_index, gene_to_pathway_map, W1, b1, W2, b2, Wp, bp,
           Wc1, bc1, Wc2, bc2):
    del gene_to_pathway_map
    src = edge_index[0]
    dst = edge_index[1]
    pad = EPAD - E
    src_p = jnp.concatenate([src, jnp.zeros((pad,), jnp.int32)])
    dst_p = jnp.concatenate([dst, jnp.full((pad,), N, jnp.int32)])
    src2d = src_p.reshape(EPAD // 128, 128)
    dst2d = dst_p.reshape(EPAD // 128, 128)
    xflat = jnp.concatenate([x[:, 0], jnp.zeros((NPAD - N,), jnp.float32)])
    xflat = xflat.reshape(1, NPAD)

    zf = jnp.zeros((NPAD,), jnp.float32)
    deg_parts = _sc_deg(dst_p, zf)
    dis, xp = _tc_combine(deg_parts, xflat)
    t_parts = _sc_t1(src_p, dst_p, xp.reshape(NPAD), zf)
    gstack = _tc_dense(t_parts, xp, dis, W1.reshape(1, H), b1.reshape(1, H),
                       W2)
    gcat = gstack.reshape(NC * NPAD, 32)
    zero_acc = jnp.zeros((NPAD, 32), jnp.bfloat16)
    acc = _sc_rowsum(src2d, dst2d, gcat, zero_acc)
    out = _tc_final(acc, gstack, dis, b2.reshape(1, H), Wp, bp.reshape(1, NPW),
                    Wc1, bc1.reshape(1, 128), Wc2, bc2.reshape(1, NCLS))
    return out


# speedup vs baseline: 64.0248x; 1.1604x over previous
import functools

import jax
import jax.numpy as jnp
from jax import lax
from jax.experimental import pallas as pl
from jax.experimental.pallas import tpu as pltpu
from jax.experimental.pallas import tpu_sc as plsc

N = 50000
E = 800000
H = 64
NPW = 100
NCLS = 5

NC, NS, L = 2, 16, 16
NW = NC * NS
NPAD = 50176
EPAD = 802816
EPW = EPAD // NW
EPT = EPAD // NS
BQ = 3136
NBQ = EPW // BQ
KSTR = 14
NBATCH = EPT // (KSTR * 128)
BCOL = 6272
TCG = NPAD // BCOL

_mesh = plsc.VectorSubcoreMesh(core_axis_name="c", subcore_axis_name="s")
_sc_params = pltpu.CompilerParams(needs_layout_passes=False)
_sc_params_nt = pltpu.CompilerParams(needs_layout_passes=False,
                                     use_tc_tiling_on_sc=False)


@functools.partial(
    pl.kernel,
    out_type=jax.ShapeDtypeStruct((NW, NPAD), jnp.float32),
    mesh=_mesh,
    compiler_params=_sc_params,
    scratch_types=[
        pltpu.VMEM((BQ,), jnp.int32),
        pltpu.VMEM((NPAD,), jnp.float32),
    ],
)
def _sc_deg(dst_hbm, zf_hbm, out_hbm, idx_v, acc_v):
    w = lax.axis_index("c") * NS + lax.axis_index("s")
    ones16 = jnp.ones((L,), jnp.float32)
    pltpu.sync_copy(zf_hbm, acc_v)
    base = w * EPW

    def chunk(q, _):
        pltpu.sync_copy(dst_hbm.at[pl.ds(base + q * BQ, BQ)], idx_v)

        def body(i, _):
            idx = idx_v[pl.ds(i * L, L)]
            plsc.addupdate_scatter(acc_v, [idx], ones16)
            return _

        lax.fori_loop(0, BQ // L, body, 0, unroll=8)
        return _

    lax.fori_loop(0, NBQ, chunk, 0)
    pltpu.sync_copy(acc_v, out_hbm.at[w])


@functools.partial(
    pl.kernel,
    out_type=jax.ShapeDtypeStruct((NW, NPAD), jnp.float32),
    mesh=_mesh,
    compiler_params=_sc_params,
    scratch_types=[
        pltpu.VMEM((BQ,), jnp.int32),
        pltpu.VMEM((BQ,), jnp.int32),
        pltpu.VMEM((NPAD,), jnp.float32),
        pltpu.VMEM((NPAD,), jnp.float32),
    ],
)
def _sc_t1(src_hbm, dst_hbm, xp_hbm, zf_hbm, out_hbm, sidx_v, didx_v, xp_v,
           acc_v):
    w = lax.axis_index("c") * NS + lax.axis_index("s")
    pltpu.sync_copy(xp_hbm, xp_v)
    pltpu.sync_copy(zf_hbm, acc_v)
    base = w * EPW

    def chunk(q, _):
        pltpu.sync_copy(src_hbm.at[pl.ds(base + q * BQ, BQ)], sidx_v)
        pltpu.sync_copy(dst_hbm.at[pl.ds(base + q * BQ, BQ)], didx_v)

        def body(i, _):
            s_idx = sidx_v[pl.ds(i * L, L)]
            vals = plsc.load_gather(xp_v, [s_idx])
            d_idx = didx_v[pl.ds(i * L, L)]
            plsc.addupdate_scatter(acc_v, [d_idx], vals)
            return _

        lax.fori_loop(0, BQ // L, body, 0, unroll=8)
        return _

    lax.fori_loop(0, NBQ, chunk, 0)
    pltpu.sync_copy(acc_v, out_hbm.at[w])


@functools.partial(
    pl.kernel,
    out_type=jax.ShapeDtypeStruct((NC, NPAD, 32), jnp.bfloat16),
    mesh=_mesh,
    compiler_params=_sc_params_nt,
    scratch_types=[
        pltpu.VMEM((2, KSTR, 128), jnp.int32),
        pltpu.VMEM((2, KSTR, 128), jnp.int32),
        [[pltpu.VMEM((128, 32), jnp.bfloat16) for _ in range(KSTR)]
         for _ in range(2)],
        pltpu.VMEM_SHARED((NPAD, 32), jnp.bfloat16),
        [pltpu.SemaphoreType.DMA for _ in range(2)],
        [pltpu.SemaphoreType.DMA for _ in range(2)],
    ],
)
def _sc_rowsum(src2d_hbm, dst2d_hbm, gcat_hbm, zero_hbm, out_hbm,
               sidx_v, didx_v, rows_v, acc_sh, gsem, ssem):
    c = lax.axis_index("c")
    s = lax.axis_index("s")

    @pl.when(s == 0)
    def _():
        pltpu.sync_copy(zero_hbm, acc_sh)

    plsc.subcore_barrier()
    coff = (c * NPAD).astype(jnp.int32)

    def drain(sem):
        for b in range(KSTR):
            pltpu.make_async_copy(
                zero_hbm.at[pl.ds(0, 128)], rows_v[0][b], sem).wait()

    def fire_gathers(k, p):
        rbase = s * (EPT // 128) + k * KSTR
        pltpu.sync_copy(src2d_hbm.at[pl.ds(rbase, KSTR)], sidx_v.at[p])
        pltpu.sync_copy(dst2d_hbm.at[pl.ds(rbase, KSTR)], didx_v.at[p])
        for b in range(KSTR):
            for q in range(128 // L):
                sidx_v[p, b, pl.ds(q * L, L)] = (
                    sidx_v[p, b, pl.ds(q * L, L)] + coff)
        for b in range(KSTR):
            pltpu.async_copy(gcat_hbm.at[sidx_v.at[p, b]], rows_v[p][b],
                             gsem[p])

    def fire_scatters(p):
        for b in range(KSTR):
            pltpu.async_copy(rows_v[p][b], acc_sh.at[didx_v.at[p, b]],
                             ssem[p], add=True)

    def pair(t, carry):
        @pl.when(t >= 1)
        def _():
            drain(ssem[0])

        fire_gathers(2 * t, 0)

        @pl.when(t >= 1)
        def _():
            drain(gsem[1])
            fire_scatters(1)
            drain(ssem[1])

        fire_gathers(2 * t + 1, 1)
        drain(gsem[0])
        fire_scatters(0)
        return carry

    lax.fori_loop(0, NBATCH // 2, pair, 0)
    drain(gsem[1])
    fire_scatters(1)
    drain(ssem[0])
    drain(ssem[1])
    plsc.subcore_barrier()
    rows_per_tile = NPAD // NS
    pltpu.sync_copy(
        acc_sh.at[pl.ds(s * rows_per_tile, rows_per_tile)],
        out_hbm.at[c, pl.ds(s * rows_per_tile, rows_per_tile)],
    )


def _tc_combine_body(p_ref, x_ref, dis_ref, xp_ref):
    deg = 1.0 + jnp.sum(p_ref[...], axis=0, keepdims=True)
    dis = 1.0 / jnp.sqrt(deg)
    dis_ref[...] = dis
    xp_ref[...] = dis * x_ref[...]


def _tc_combine(p, xflat):
    return pl.pallas_call(
        _tc_combine_body,
        grid=(TCG,),
        in_specs=[
            pl.BlockSpec((NW, BCOL), lambda i: (0, i)),
            pl.BlockSpec((1, BCOL), lambda i: (0, i)),
        ],
        out_specs=[
            pl.BlockSpec((1, BCOL), lambda i: (0, i)),
            pl.BlockSpec((1, BCOL), lambda i: (0, i)),
        ],
        out_shape=[
            jax.ShapeDtypeStruct((1, NPAD), jnp.float32),
            jax.ShapeDtypeStruct((1, NPAD), jnp.float32),
        ],
    )(p, xflat)


def _tc_dense_body(t_ref, xp_ref, dis_ref, w1_ref, b1_ref, w2_ref, g_ref):
    t1_raw = jnp.sum(t_ref[...], axis=0, keepdims=True) + xp_ref[...]
    t1 = dis_ref[...] * t1_raw
    h1p = lax.dot_general(t1, w1_ref[...], (((0,), (0,)), ((), ())),
                          preferred_element_type=jnp.float32)
    ones_row = jnp.ones((1, H), jnp.float32)
    d_mat = lax.dot_general(dis_ref[...], ones_row, (((0,), (0,)), ((), ())),
                            preferred_element_type=jnp.float32)
    h1 = jnp.maximum(h1p + b1_ref[...], 0.0)
    g = lax.dot_general(h1, w2_ref[...], (((1,), (0,)), ((), ())),
                        preferred_element_type=jnp.float32)
    gp = (d_mat * g).astype(jnp.bfloat16)
    g_ref[0] = gp[:, :32]
    g_ref[1] = gp[:, 32:]


def _tc_dense(t, xp, dis, w1, b1, w2):
    return pl.pallas_call(
        _tc_dense_body,
        grid=(TCG,),
        in_specs=[
            pl.BlockSpec((NW, BCOL), lambda i: (0, i)),
            pl.BlockSpec((1, BCOL), lambda i: (0, i)),
            pl.BlockSpec((1, BCOL), lambda i: (0, i)),
            pl.BlockSpec((1, H), lambda i: (0, 0)),
            pl.BlockSpec((1, H), lambda i: (0, 0)),
            pl.BlockSpec((H, H), lambda i: (0, 0)),
        ],
        out_specs=pl.BlockSpec((NC, BCOL, 32), lambda i: (0, i, 0)),
        out_shape=jax.ShapeDtypeStruct((NC, NPAD, 32), jnp.bfloat16),
    )(t, xp, dis, w1, b1, w2)


def _tc_final_body(acc_ref, g_ref, dis_ref, b2_ref, wp_ref, bp_ref,
                   wc1_ref, bc1_ref, wc2_ref, bc2_ref, out_ref, s_acc):
    i = pl.program_id(0)

    @pl.when(i == 0)
    def _():
        s_acc[...] = jnp.zeros_like(s_acc)

    ones_row = jnp.ones((1, H), jnp.float32)
    d_mat = lax.dot_general(dis_ref[...], ones_row, (((0,), (0,)), ((), ())),
                            preferred_element_type=jnp.float32)
    accf = jnp.concatenate([acc_ref[0], acc_ref[1]],
                           axis=1).astype(jnp.float32)
    gf = jnp.concatenate([g_ref[0], g_ref[1]],
                         axis=1).astype(jnp.float32)
    agg = d_mat * (accf + gf) + b2_ref[...]
    h2 = jnp.maximum(agg, 0.0)
    row = lax.broadcasted_iota(jnp.int32, (BCOL, H), 0) + i * BCOL
    h2 = jnp.where(row < N, h2, 0.0)
    pf = lax.dot_general(h2, wp_ref[...], (((1,), (0,)), ((), ())),
                         preferred_element_type=jnp.float32)
    s_acc[...] = s_acc[...] + jnp.sum(pf, axis=0, keepdims=True)

    @pl.when(i == TCG - 1)
    def _():
        pooled = s_acc[...] * (1.0 / N) + bp_ref[...]
        z = jnp.maximum(
            lax.dot_general(pooled, wc1_ref[...], (((1,), (0,)), ((), ())),
                            preferred_element_type=jnp.float32) + bc1_ref[...],
            0.0)
        out_ref[...] = lax.dot_general(
            z, wc2_ref[...], (((1,), (0,)), ((), ())),
            preferred_element_type=jnp.float32) + bc2_ref[...]


def _tc_final(acc, g, dis, b2, wp, bp, wc1, bc1, wc2, bc2):
    return pl.pallas_call(
        _tc_final_body,
        grid=(TCG,),
        in_specs=[
            pl.BlockSpec((NC, BCOL, 32), lambda i: (0, i, 0)),
            pl.BlockSpec((NC, BCOL, 32), lambda i: (0, i, 0)),
            pl.BlockSpec((1, BCOL), lambda i: (0, i)),
            pl.BlockSpec((1, H), lambda i: (0, 0)),
            pl.BlockSpec((H, NPW), lambda i: (0, 0)),
            pl.BlockSpec((1, NPW), lambda i: (0, 0)),
            pl.BlockSpec((NPW, 128), lambda i: (0, 0)),
            pl.BlockSpec((1, 128), lambda i: (0, 0)),
            pl.BlockSpec((128, NCLS), lambda i: (0, 0)),
            pl.BlockSpec((1, NCLS), lambda i: (0, 0)),
        ],
        out_specs=pl.BlockSpec((1, NCLS), lambda i: (0, 0)),
        out_shape=jax.ShapeDtypeStruct((1, NCLS), jnp.float32),
        scratch_shapes=[pltpu.VMEM((1, NPW), jnp.float32)],
    )(acc, g, dis, b2, wp, bp, wc1, bc1, wc2, bc2)


def kernel(x, edge_index, gene_to_pathway_map, W1, b1, W2, b2, Wp, bp,
           Wc1, bc1, Wc2, bc2):
    del gene_to_pathway_map
    src = edge_index[0]
    dst = edge_index[1]
    pad = EPAD - E
    src_p = jnp.concatenate([src, jnp.zeros((pad,), jnp.int32)])
    dst_p = jnp.concatenate([dst, jnp.full((pad,), N, jnp.int32)])
    src2d = src_p.reshape(EPAD // 128, 128)
    dst2d = dst_p.reshape(EPAD // 128, 128)
    xflat = jnp.concatenate([x[:, 0], jnp.zeros((NPAD - N,), jnp.float32)])
    xflat = xflat.reshape(1, NPAD)

    zf = jnp.zeros((NPAD,), jnp.float32)
    deg_parts = _sc_deg(dst_p, zf)
    dis, xp = _tc_combine(deg_parts, xflat)
    t_parts = _sc_t1(src_p, dst_p, xp.reshape(NPAD), zf)
    gstack = _tc_dense(t_parts, xp, dis, W1.reshape(1, H), b1.reshape(1, H),
                       W2)
    gcat = gstack.reshape(NC * NPAD, 32)
    zero_acc = jnp.zeros((NPAD, 32), jnp.bfloat16)
    acc = _sc_rowsum(src2d, dst2d, gcat, zero_acc)
    out = _tc_final(acc, gstack, dis, b2.reshape(1, H), Wp, bp.reshape(1, NPW),
                    Wc1, bc1.reshape(1, 128), Wc2, bc2.reshape(1, NCLS))
    return out


# speedup vs baseline: 65.6379x; 1.0252x over previous
import functools

import jax
import jax.numpy as jnp
from jax import lax
from jax.experimental import pallas as pl
from jax.experimental.pallas import tpu as pltpu
from jax.experimental.pallas import tpu_sc as plsc

N = 50000
E = 800000
H = 64
NPW = 100
NCLS = 5

NC, NS, L = 2, 16, 16
NW = NC * NS
NPAD = 50176
EPAD = 802816
EPW = EPAD // NW
EPT = EPAD // NS
BQA = 25088
BQB = 12544
NBQB = EPW // BQB
KSTR = 14
NBATCH = EPT // (KSTR * 128)
BCOL = 6272
TCG = NPAD // BCOL

_mesh = plsc.VectorSubcoreMesh(core_axis_name="c", subcore_axis_name="s")
_sc_params = pltpu.CompilerParams(needs_layout_passes=False)
_sc_params_nt = pltpu.CompilerParams(needs_layout_passes=False,
                                     use_tc_tiling_on_sc=False)


@functools.partial(
    pl.kernel,
    out_type=jax.ShapeDtypeStruct((NW, NPAD), jnp.float32),
    mesh=_mesh,
    compiler_params=_sc_params,
    scratch_types=[
        pltpu.VMEM((BQA,), jnp.int32),
        pltpu.VMEM((NPAD,), jnp.float32),
    ],
)
def _sc_deg(dst_hbm, zf_hbm, out_hbm, idx_v, acc_v):
    w = lax.axis_index("c") * NS + lax.axis_index("s")
    ones16 = jnp.ones((L,), jnp.float32)
    pltpu.sync_copy(zf_hbm, acc_v)
    base = w * EPW

    pltpu.sync_copy(dst_hbm.at[pl.ds(base, BQA)], idx_v)

    def body(i, _):
        idx = idx_v[pl.ds(i * L, L)]
        plsc.addupdate_scatter(acc_v, [idx], ones16)
        return _

    lax.fori_loop(0, BQA // L, body, 0, unroll=8)
    pltpu.sync_copy(acc_v, out_hbm.at[w])


@functools.partial(
    pl.kernel,
    out_type=jax.ShapeDtypeStruct((NW, NPAD), jnp.float32),
    mesh=_mesh,
    compiler_params=_sc_params,
    scratch_types=[
        pltpu.VMEM((BQB,), jnp.int32),
        pltpu.VMEM((BQB,), jnp.int32),
        pltpu.VMEM((NPAD,), jnp.float32),
        pltpu.VMEM((NPAD,), jnp.float32),
    ],
)
def _sc_t1(src_hbm, dst_hbm, xp_hbm, zf_hbm, out_hbm, sidx_v, didx_v, xp_v,
           acc_v):
    w = lax.axis_index("c") * NS + lax.axis_index("s")
    pltpu.sync_copy(xp_hbm, xp_v)
    pltpu.sync_copy(zf_hbm, acc_v)
    base = w * EPW

    def chunk(q, _):
        pltpu.sync_copy(src_hbm.at[pl.ds(base + q * BQB, BQB)], sidx_v)
        pltpu.sync_copy(dst_hbm.at[pl.ds(base + q * BQB, BQB)], didx_v)

        def body(i, _):
            s_idx = sidx_v[pl.ds(i * L, L)]
            vals = plsc.load_gather(xp_v, [s_idx])
            d_idx = didx_v[pl.ds(i * L, L)]
            plsc.addupdate_scatter(acc_v, [d_idx], vals)
            return _

        lax.fori_loop(0, BQB // L, body, 0, unroll=8)
        return _

    lax.fori_loop(0, NBQB, chunk, 0)
    pltpu.sync_copy(acc_v, out_hbm.at[w])


@functools.partial(
    pl.kernel,
    out_type=jax.ShapeDtypeStruct((NC, NPAD, 32), jnp.bfloat16),
    mesh=_mesh,
    compiler_params=_sc_params_nt,
    scratch_types=[
        pltpu.VMEM((2, KSTR, 128), jnp.int32),
        pltpu.VMEM((2, KSTR, 128), jnp.int32),
        [[pltpu.VMEM((128, 32), jnp.bfloat16) for _ in range(KSTR)]
         for _ in range(2)],
        pltpu.VMEM_SHARED((NPAD, 32), jnp.bfloat16),
        [pltpu.SemaphoreType.DMA for _ in range(2)],
        [pltpu.SemaphoreType.DMA for _ in range(2)],
    ],
)
def _sc_rowsum(src2d_hbm, dst2d_hbm, gcat_hbm, zero_hbm, out_hbm,
               sidx_v, didx_v, rows_v, acc_sh, gsem, ssem):
    c = lax.axis_index("c")
    s = lax.axis_index("s")

    @pl.when(s == 0)
    def _():
        pltpu.sync_copy(zero_hbm, acc_sh)

    plsc.subcore_barrier()
    coff = (c * NPAD).astype(jnp.int32)

    def drain(sem):
        for b in range(KSTR):
            pltpu.make_async_copy(
                zero_hbm.at[pl.ds(0, 128)], rows_v[0][b], sem).wait()

    def fire_gathers(k, p):
        rbase = s * (EPT // 128) + k * KSTR
        pltpu.sync_copy(src2d_hbm.at[pl.ds(rbase, KSTR)], sidx_v.at[p])
        pltpu.sync_copy(dst2d_hbm.at[pl.ds(rbase, KSTR)], didx_v.at[p])
        for b in range(KSTR):
            for q in range(128 // L):
                sidx_v[p, b, pl.ds(q * L, L)] = (
                    sidx_v[p, b, pl.ds(q * L, L)] + coff)
        for b in range(KSTR):
            pltpu.async_copy(gcat_hbm.at[sidx_v.at[p, b]], rows_v[p][b],
                             gsem[p])

    def fire_scatters(p):
        for b in range(KSTR):
            pltpu.async_copy(rows_v[p][b], acc_sh.at[didx_v.at[p, b]],
                             ssem[p], add=True)

    def pair(t, carry):
        @pl.when(t >= 1)
        def _():
            drain(ssem[0])

        fire_gathers(2 * t, 0)

        @pl.when(t >= 1)
        def _():
            drain(gsem[1])
            fire_scatters(1)
            drain(ssem[1])

        fire_gathers(2 * t + 1, 1)
        drain(gsem[0])
        fire_scatters(0)
        return carry

    lax.fori_loop(0, NBATCH // 2, pair, 0)
    drain(gsem[1])
    fire_scatters(1)
    drain(ssem[0])
    drain(ssem[1])
    plsc.subcore_barrier()
    rows_per_tile = NPAD // NS
    pltpu.sync_copy(
        acc_sh.at[pl.ds(s * rows_per_tile, rows_per_tile)],
        out_hbm.at[c, pl.ds(s * rows_per_tile, rows_per_tile)],
    )


def _tc_combine_body(p_ref, x_ref, dis_ref, xp_ref):
    deg = 1.0 + jnp.sum(p_ref[...], axis=0, keepdims=True)
    dis = 1.0 / jnp.sqrt(deg)
    dis_ref[...] = dis
    xp_ref[...] = dis * x_ref[...]


def _tc_combine(p, xflat):
    return pl.pallas_call(
        _tc_combine_body,
        grid=(TCG,),
        in_specs=[
            pl.BlockSpec((NW, BCOL), lambda i: (0, i)),
            pl.BlockSpec((1, BCOL), lambda i: (0, i)),
        ],
        out_specs=[
            pl.BlockSpec((1, BCOL), lambda i: (0, i)),
            pl.BlockSpec((1, BCOL), lambda i: (0, i)),
        ],
        out_shape=[
            jax.ShapeDtypeStruct((1, NPAD), jnp.float32),
            jax.ShapeDtypeStruct((1, NPAD), jnp.float32),
        ],
    )(p, xflat)


def _tc_dense_body(t_ref, xp_ref, dis_ref, w1_ref, b1_ref, w2_ref, g_ref):
    t1_raw = jnp.sum(t_ref[...], axis=0, keepdims=True) + xp_ref[...]
    t1 = dis_ref[...] * t1_raw
    h1p = lax.dot_general(t1, w1_ref[...], (((0,), (0,)), ((), ())),
                          preferred_element_type=jnp.float32)
    ones_row = jnp.ones((1, H), jnp.float32)
    d_mat = lax.dot_general(dis_ref[...], ones_row, (((0,), (0,)), ((), ())),
                            preferred_element_type=jnp.float32)
    h1 = jnp.maximum(h1p + b1_ref[...], 0.0)
    g = lax.dot_general(h1, w2_ref[...], (((1,), (0,)), ((), ())),
                        preferred_element_type=jnp.float32)
    gp = (d_mat * g).astype(jnp.bfloat16)
    g_ref[0] = gp[:, :32]
    g_ref[1] = gp[:, 32:]


def _tc_dense(t, xp, dis, w1, b1, w2):
    return pl.pallas_call(
        _tc_dense_body,
        grid=(TCG,),
        in_specs=[
            pl.BlockSpec((NW, BCOL), lambda i: (0, i)),
            pl.BlockSpec((1, BCOL), lambda i: (0, i)),
            pl.BlockSpec((1, BCOL), lambda i: (0, i)),
            pl.BlockSpec((1, H), lambda i: (0, 0)),
            pl.BlockSpec((1, H), lambda i: (0, 0)),
            pl.BlockSpec((H, H), lambda i: (0, 0)),
        ],
        out_specs=pl.BlockSpec((NC, BCOL, 32), lambda i: (0, i, 0)),
        out_shape=jax.ShapeDtypeStruct((NC, NPAD, 32), jnp.bfloat16),
    )(t, xp, dis, w1, b1, w2)


def _tc_final_body(acc_ref, g_ref, dis_ref, b2_ref, wp_ref, bp_ref,
                   wc1_ref, bc1_ref, wc2_ref, bc2_ref, out_ref, s_acc):
    i = pl.program_id(0)

    @pl.when(i == 0)
    def _():
        s_acc[...] = jnp.zeros_like(s_acc)

    ones_row = jnp.ones((1, H), jnp.float32)
    d_mat = lax.dot_general(dis_ref[...], ones_row, (((0,), (0,)), ((), ())),
                            preferred_element_type=jnp.float32)
    accf = jnp.concatenate([acc_ref[0], acc_ref[1]],
                           axis=1).astype(jnp.float32)
    gf = jnp.concatenate([g_ref[0], g_ref[1]],
                         axis=1).astype(jnp.float32)
    agg = d_mat * (accf + gf) + b2_ref[...]
    h2 = jnp.maximum(agg, 0.0)
    row = lax.broadcasted_iota(jnp.int32, (BCOL, H), 0) + i * BCOL
    h2 = jnp.where(row < N, h2, 0.0)
    pf = lax.dot_general(h2, wp_ref[...], (((1,), (0,)), ((), ())),
                         preferred_element_type=jnp.float32)
    s_acc[...] = s_acc[...] + jnp.sum(pf, axis=0, keepdims=True)

    @pl.when(i == TCG - 1)
    def _():
        pooled = s_acc[...] * (1.0 / N) + bp_ref[...]
        z = jnp.maximum(
            lax.dot_general(pooled, wc1_ref[...], (((1,), (0,)), ((), ())),
                            preferred_element_type=jnp.float32) + bc1_ref[...],
            0.0)
        out_ref[...] = lax.dot_general(
            z, wc2_ref[...], (((1,), (0,)), ((), ())),
            preferred_element_type=jnp.float32) + bc2_ref[...]


def _tc_final(acc, g, dis, b2, wp, bp, wc1, bc1, wc2, bc2):
    return pl.pallas_call(
        _tc_final_body,
        grid=(TCG,),
        in_specs=[
            pl.BlockSpec((NC, BCOL, 32), lambda i: (0, i, 0)),
            pl.BlockSpec((NC, BCOL, 32), lambda i: (0, i, 0)),
            pl.BlockSpec((1, BCOL), lambda i: (0, i)),
            pl.BlockSpec((1, H), lambda i: (0, 0)),
            pl.BlockSpec((H, NPW), lambda i: (0, 0)),
            pl.BlockSpec((1, NPW), lambda i: (0, 0)),
            pl.BlockSpec((NPW, 128), lambda i: (0, 0)),
            pl.BlockSpec((1, 128), lambda i: (0, 0)),
            pl.BlockSpec((128, NCLS), lambda i: (0, 0)),
            pl.BlockSpec((1, NCLS), lambda i: (0, 0)),
        ],
        out_specs=pl.BlockSpec((1, NCLS), lambda i: (0, 0)),
        out_shape=jax.ShapeDtypeStruct((1, NCLS), jnp.float32),
        scratch_shapes=[pltpu.VMEM((1, NPW), jnp.float32)],
    )(acc, g, dis, b2, wp, bp, wc1, bc1, wc2, bc2)


def kernel(x, edge_index, gene_to_pathway_map, W1, b1, W2, b2, Wp, bp,
           Wc1, bc1, Wc2, bc2):
    del gene_to_pathway_map
    src = edge_index[0]
    dst = edge_index[1]
    pad = EPAD - E
    src_p = jnp.concatenate([src, jnp.zeros((pad,), jnp.int32)])
    dst_p = jnp.concatenate([dst, jnp.full((pad,), N, jnp.int32)])
    src2d = src_p.reshape(EPAD // 128, 128)
    dst2d = dst_p.reshape(EPAD // 128, 128)
    xflat = jnp.concatenate([x[:, 0], jnp.zeros((NPAD - N,), jnp.float32)])
    xflat = xflat.reshape(1, NPAD)

    zf = jnp.zeros((NPAD,), jnp.float32)
    deg_parts = _sc_deg(dst_p, zf)
    dis, xp = _tc_combine(deg_parts, xflat)
    t_parts = _sc_t1(src_p, dst_p, xp.reshape(NPAD), zf)
    gstack = _tc_dense(t_parts, xp, dis, W1.reshape(1, H), b1.reshape(1, H),
                       W2)
    gcat = gstack.reshape(NC * NPAD, 32)
    zero_acc = jnp.zeros((NPAD, 32), jnp.bfloat16)
    acc = _sc_rowsum(src2d, dst2d, gcat, zero_acc)
    out = _tc_final(acc, gstack, dis, b2.reshape(1, H), Wp, bp.reshape(1, NPW),
                    Wc1, bc1.reshape(1, 128), Wc2, bc2.reshape(1, NCLS))
    return out
